# baseline (speedup 1.0000x reference)
import functools

import jax
import jax.numpy as jnp
from jax import lax
from jax.experimental import pallas as pl
from jax.experimental.pallas import tpu as pltpu
from jax.experimental.pallas import tpu_sc as plsc

_B, _N, _D, _E, _A = 32, 128, 256, 16, 4096
_L = 16
_NCHUNK = _A // _L


def _tc_body(he_ref, he2_ref, h_ref, wedit_ref, bedit_ref, wstop_ref,
             bstop_ref, t4_ref, s4t_ref, stop_ref):
    h = h_ref[0]
    si = jnp.dot(h, wedit_ref[:_D], preferred_element_type=jnp.float32)
    si = si + bedit_ref[...]
    sj = jnp.dot(h, wedit_ref[_D:2 * _D], preferred_element_type=jnp.float32)
    stop_col = jnp.dot(h, wstop_ref[...], preferred_element_type=jnp.float32)
    stop_ref[...] = jnp.full(
        (1, 1, 128),
        jnp.sum(stop_col) * (1.0 / _N) + bstop_ref[0, 0], jnp.float32)
    s4t_ref[0, 0:4] = si.T
    s4t_ref[0, 4:8] = sj.T
    we_t = wedit_ref[2 * _D:].T
    rows = lax.broadcasted_iota(jnp.int32, (32, 128), 0)
    cols = lax.broadcasted_iota(jnp.int32, (32, 128), 1)
    w4t = jnp.where(rows // 4 == cols // _E, jnp.tile(we_t, (8, 8)), 0.0)
    sjt = sj.T
    sj32 = jnp.broadcast_to(sjt[None, :, :], (8, 4, _N)).reshape(32, _N)
    for g in range(_N // 8):
        src = he_ref if g < 8 else he2_ref
        m = src[0, pl.ds(8 * g % 64, 8)].reshape(128, _N)
        out = jnp.dot(w4t, m, preferred_element_type=jnp.float32) + sj32
        t4_ref[0, pl.ds(8 * g, 8)] = out.reshape(8, 4, _N)


def _tc_tables(he_t, h_nodes, w_edit, b_edit, w_stop, b_stop):
    return pl.pallas_call(
        _tc_body,
        grid=(_B,),
        in_specs=[
            pl.BlockSpec((1, _N // 2, _E, _N), lambda b: (b, 0, 0, 0)),
            pl.BlockSpec((1, _N // 2, _E, _N), lambda b: (b, 1, 0, 0)),
            pl.BlockSpec((1, _N, _D), lambda b: (b, 0, 0)),
            pl.BlockSpec((2 * _D + _E, 4), lambda b: (0, 0)),
            pl.BlockSpec((1, 4), lambda b: (0, 0)),
            pl.BlockSpec((_D, 1), lambda b: (0, 0)),
            pl.BlockSpec((1, 1), lambda b: (0, 0)),
        ],
        out_specs=[
            pl.BlockSpec((1, _N, 4, _N), lambda b: (b, 0, 0, 0)),
            pl.BlockSpec((1, 8, _N), lambda b: (b, 0, 0)),
            pl.BlockSpec((1, 1, 128), lambda b: (b, 0, 0)),
        ],
        out_shape=[
            jax.ShapeDtypeStruct((_B, _N, 4, _N), jnp.float32),
            jax.ShapeDtypeStruct((_B, 8, _N), jnp.float32),
            jax.ShapeDtypeStruct((_B, 1, 128), jnp.float32),
        ],
    )(he_t, he_t, h_nodes, w_edit, b_edit, w_stop, b_stop)


def _sc_edit_logits(t4, s4t, i_idx, j_idx, c_idx):
    info = plsc.get_sparse_core_info()
    mesh = plsc.VectorSubcoreMesh(core_axis_name="c", subcore_axis_name="s")

    @functools.partial(
        pl.kernel,
        out_type=jax.ShapeDtypeStruct((_B, _A // 128, 128), jnp.float32),
        mesh=mesh,
        compiler_params=pltpu.CompilerParams(
            needs_layout_passes=False, use_tc_tiling_on_sc=False),
        scratch_types=[
            pltpu.VMEM((_A,), jnp.int32),
            pltpu.VMEM((_A,), jnp.int32),
            pltpu.VMEM((_A,), jnp.int32),
            pltpu.VMEM((_N, 4, _N), jnp.float32),
            pltpu.VMEM((8, _N), jnp.float32),
            pltpu.VMEM((_A // 128, 128), jnp.float32),
            pltpu.SemaphoreType.DMA,
        ],
    )
    def k(t4_hbm, s4t_hbm, i_hbm, j_hbm, c_hbm,
          out_hbm, i_v, j_v, c_v, t4_v, s4t_v, out_v, sem):
        wid = lax.axis_index("s") * info.num_cores + lax.axis_index("c")
        cps = [
            pltpu.async_copy(i_hbm, i_v, sem),
            pltpu.async_copy(j_hbm, j_v, sem),
            pltpu.async_copy(c_hbm, c_v, sem),
            pltpu.async_copy(t4_hbm.at[wid], t4_v, sem),
            pltpu.async_copy(s4t_hbm.at[wid], s4t_v, sem),
        ]
        for cp in cps:
            cp.wait()

        @plsc.parallel_loop(0, _NCHUNK, unroll=4)
        def chunk(q):
            a0 = q * _L
            iv = i_v[pl.ds(a0, _L)]
            jv = j_v[pl.ds(a0, _L)]
            cv = c_v[pl.ds(a0, _L)]
            acc = plsc.load_gather(t4_v, [iv, cv, jv])
            acc = acc + plsc.load_gather(s4t_v, [cv, iv])
            out_v[q // 8, pl.ds((q % 8) * _L, _L)] = acc

        pltpu.sync_copy(out_v, out_hbm.at[wid])

    return k(t4, s4t, i_idx, j_idx, c_idx)


def kernel(h_nodes, h_edges, W_edit, b_edit, W_stop, b_stop, edit_ij, edit_b,
           feas, stop_feas):
    he_t = jnp.transpose(h_edges, (0, 1, 3, 2))

    t4, s4t, stop_tab = _tc_tables(he_t, h_nodes, W_edit, b_edit[None, :],
                                   W_stop, b_stop[None, :])
    raw = _sc_edit_logits(t4, s4t, edit_ij[0], edit_ij[1], edit_b)

    minf = jnp.float32(-jnp.inf)
    edit_logits = jnp.where(feas.astype(bool), raw.reshape(_B, _A), minf)
    stop = jnp.where(stop_feas.astype(bool), stop_tab[:, 0, 0], minf)
    return jnp.concatenate([edit_logits, stop[:, None]], axis=1)


# speedup vs baseline: 1.0376x; 1.0376x over previous
import functools

import jax
import jax.numpy as jnp
from jax import lax
from jax.experimental import pallas as pl
from jax.experimental.pallas import tpu as pltpu
from jax.experimental.pallas import tpu_sc as plsc

_B, _N, _D, _E, _A = 32, 128, 256, 16, 4096
_L = 16
_NCHUNK = _A // _L


def _tc_body(he_ref, he2_ref, h_ref, wedit_ref, bedit_ref, wstop_ref,
             bstop_ref, t4_ref, s4t_ref, stop_ref):
    h = h_ref[0]
    si = jnp.dot(h, wedit_ref[:_D], preferred_element_type=jnp.float32)
    si = si + bedit_ref[...]
    sj = jnp.dot(h, wedit_ref[_D:2 * _D], preferred_element_type=jnp.float32)
    stop_col = jnp.dot(h, wstop_ref[...], preferred_element_type=jnp.float32)
    stop_ref[...] = jnp.full(
        (1, 1, 128),
        jnp.sum(stop_col) * (1.0 / _N) + bstop_ref[0, 0], jnp.float32)
    s4t_ref[0, 0:4] = si.T
    s4t_ref[0, 4:8] = sj.T
    we_t = wedit_ref[2 * _D:].T
    rows = lax.broadcasted_iota(jnp.int32, (32, 128), 0)
    cols = lax.broadcasted_iota(jnp.int32, (32, 128), 1)
    w4t = jnp.where(rows // 4 == cols // _E, jnp.tile(we_t, (8, 8)), 0.0)
    sjt = sj.T
    sj32 = jnp.broadcast_to(sjt[None, :, :], (8, 4, _N)).reshape(32, _N)
    for g in range(_N // 8):
        src = he_ref if g < 8 else he2_ref
        m = src[0, pl.ds(8 * g % 64, 8)].reshape(128, _N)
        out = jnp.dot(w4t, m, preferred_element_type=jnp.float32) + sj32
        outr = out.astype(jnp.bfloat16).astype(jnp.float32)
        bits = jax.lax.bitcast_convert_type(outr, jnp.int32)
        word = bits[16:32] | jax.lax.shift_right_logical(bits[0:16], 16)
        pk = jax.lax.bitcast_convert_type(word, jnp.float32)
        t4_ref[0, pl.ds(16 * g, 16)] = pk


def _tc_tables(he_t, h_nodes, w_edit, b_edit, w_stop, b_stop):
    return pl.pallas_call(
        _tc_body,
        grid=(_B,),
        in_specs=[
            pl.BlockSpec((1, _N // 2, _E, _N), lambda b: (b, 0, 0, 0)),
            pl.BlockSpec((1, _N // 2, _E, _N), lambda b: (b, 1, 0, 0)),
            pl.BlockSpec((1, _N, _D), lambda b: (b, 0, 0)),
            pl.BlockSpec((2 * _D + _E, 4), lambda b: (0, 0)),
            pl.BlockSpec((1, 4), lambda b: (0, 0)),
            pl.BlockSpec((_D, 1), lambda b: (0, 0)),
            pl.BlockSpec((1, 1), lambda b: (0, 0)),
        ],
        out_specs=[
            pl.BlockSpec((1, _N * 2, _N), lambda b: (b, 0, 0)),
            pl.BlockSpec((1, 8, _N), lambda b: (b, 0, 0)),
            pl.BlockSpec((1, 1, 128), lambda b: (b, 0, 0)),
        ],
        out_shape=[
            jax.ShapeDtypeStruct((_B, _N * 2, _N), jnp.float32),
            jax.ShapeDtypeStruct((_B, 8, _N), jnp.float32),
            jax.ShapeDtypeStruct((_B, 1, 128), jnp.float32),
        ],
    )(he_t, he_t, h_nodes, w_edit, b_edit, w_stop, b_stop)


def _sc_edit_logits(t4, s4t, i_idx, j_idx, c_idx):
    info = plsc.get_sparse_core_info()
    mesh = plsc.VectorSubcoreMesh(core_axis_name="c", subcore_axis_name="s")

    @functools.partial(
        pl.kernel,
        out_type=jax.ShapeDtypeStruct((_B, _A // 128, 128), jnp.float32),
        mesh=mesh,
        compiler_params=pltpu.CompilerParams(
            needs_layout_passes=False, use_tc_tiling_on_sc=False),
        scratch_types=[
            pltpu.VMEM((_A,), jnp.int32),
            pltpu.VMEM((_A,), jnp.int32),
            pltpu.VMEM((_A,), jnp.int32),
            pltpu.VMEM((_N * 2, _N), jnp.float32),
            pltpu.VMEM((8, _N), jnp.float32),
            pltpu.VMEM((_A // 128, 128), jnp.float32),
            pltpu.SemaphoreType.DMA,
        ],
    )
    def k(t4_hbm, s4t_hbm, i_hbm, j_hbm, c_hbm,
          out_hbm, i_v, j_v, c_v, t4_v, s4t_v, out_v, sem):
        wid = lax.axis_index("s") * info.num_cores + lax.axis_index("c")
        cps = [
            pltpu.async_copy(i_hbm, i_v, sem),
            pltpu.async_copy(j_hbm, j_v, sem),
            pltpu.async_copy(c_hbm, c_v, sem),
            pltpu.async_copy(t4_hbm.at[wid], t4_v, sem),
            pltpu.async_copy(s4t_hbm.at[wid], s4t_v, sem),
        ]
        for cp in cps:
            cp.wait()

        mhi = jnp.int32(-65536)

        @plsc.parallel_loop(0, _NCHUNK, unroll=4)
        def chunk(q):
            a0 = q * _L
            iv = i_v[pl.ds(a0, _L)]
            jv = j_v[pl.ds(a0, _L)]
            cv = c_v[pl.ds(a0, _L)]
            r = ((iv & 7) << 2) + cv
            row = ((iv >> 3) << 4) + (r & 15)
            word = plsc.load_gather(t4_v, [row, jv])
            bits = plsc.bitcast(word, jnp.int32)
            vbits = jnp.where(r >= 16, bits & mhi, bits << 16)
            acc = plsc.bitcast(vbits, jnp.float32)
            acc = acc + plsc.load_gather(s4t_v, [cv, iv])
            out_v[q // 8, pl.ds((q % 8) * _L, _L)] = acc

        pltpu.sync_copy(out_v, out_hbm.at[wid])

    return k(t4, s4t, i_idx, j_idx, c_idx)


def kernel(h_nodes, h_edges, W_edit, b_edit, W_stop, b_stop, edit_ij, edit_b,
           feas, stop_feas):
    he_t = jnp.transpose(h_edges, (0, 1, 3, 2))

    t4, s4t, stop_tab = _tc_tables(he_t, h_nodes, W_edit, b_edit[None, :],
                                   W_stop, b_stop[None, :])
    raw = _sc_edit_logits(t4, s4t, edit_ij[0], edit_ij[1], edit_b)

    minf = jnp.float32(-jnp.inf)
    edit_logits = jnp.where(feas.astype(bool), raw.reshape(_B, _A), minf)
    stop = jnp.where(stop_feas.astype(bool), stop_tab[:, 0, 0], minf)
    return jnp.concatenate([edit_logits, stop[:, None]], axis=1)
